# Initial kernel scaffold; baseline (speedup 1.0000x reference)
import jax
import jax.numpy as jnp
from jax.experimental import pallas as pl


def kernel(data_in, octree):
    raise NotImplementedError("write your pallas kernel here")


# speedup vs baseline: 2.3406x; 2.3406x over previous
import functools

import jax
import jax.numpy as jnp
from jax import lax
from jax.experimental import pallas as pl
from jax.experimental.pallas import tpu as pltpu
from jax.experimental.pallas import tpu_sc as plsc

_INFO = plsc.get_sparse_core_info()
_NC, _NS, _L = _INFO.num_cores, _INFO.num_subcores, _INFO.num_lanes

_CHUNK = 2048


@functools.partial(jax.jit, static_argnums=(2, 3, 4))
def _col2octree_sc(data2, idx, C, K, H):
    J = K * H
    n_chunks = J // _CHUNK

    mesh = plsc.VectorSubcoreMesh(core_axis_name="c", subcore_axis_name="s")

    @functools.partial(
        pl.kernel,
        mesh=mesh,
        out_type=jax.ShapeDtypeStruct((C, H), jnp.float32),
        compiler_params=pltpu.CompilerParams(needs_layout_passes=False),
        scratch_types=[
            pltpu.VMEM((H,), jnp.float32),
            pltpu.VMEM((_CHUNK,), jnp.int32),
            pltpu.VMEM((_CHUNK,), jnp.float32),
        ],
    )
    def k(data_hbm, idx_hbm, out_hbm, accum, idxb, datab):
        ch = lax.axis_index("s") * _NC + lax.axis_index("c")

        zeros = jnp.zeros((_L,), jnp.float32)

        def zbody(i, carry):
            accum[pl.ds(i * _L, _L)] = zeros
            return carry

        lax.fori_loop(0, H // _L, zbody, 0)

        def cbody(g, carry):
            off = g * _CHUNK
            pltpu.sync_copy(idx_hbm.at[pl.ds(off, _CHUNK)], idxb)
            pltpu.sync_copy(data_hbm.at[ch, pl.ds(off, _CHUNK)], datab)

            def ibody(i, icarry):
                vi = idxb[pl.ds(i * _L, _L)]
                vd = datab[pl.ds(i * _L, _L)]
                plsc.addupdate_scatter(accum, [vi], vd)
                return icarry

            lax.fori_loop(0, _CHUNK // _L, ibody, 0)
            return carry

        lax.fori_loop(0, n_chunks, cbody, 0)
        pltpu.sync_copy(accum, out_hbm.at[ch])

    return k(data2, idx)


def kernel(data_in, octree):
    C, K, H = data_in.shape
    idx = octree.T.reshape(K * H)
    data2 = data_in.reshape(C, K * H)
    return _col2octree_sc(data2, idx, C, K, H)


# speedup vs baseline: 4.6934x; 2.0052x over previous
import functools

import jax
import jax.numpy as jnp
from jax import lax
from jax.experimental import pallas as pl
from jax.experimental.pallas import tpu as pltpu
from jax.experimental.pallas import tpu_sc as plsc

_INFO = plsc.get_sparse_core_info()
_NC, _NS, _L = _INFO.num_cores, _INFO.num_subcores, _INFO.num_lanes

_CHUNK = 4096
_NBUF = 2
_UNROLL = 8


@functools.partial(jax.jit, static_argnums=(2, 3, 4))
def _col2octree_sc(data2, idx, C, K, H):
    J = K * H
    n_chunks = J // _CHUNK

    mesh = plsc.VectorSubcoreMesh(core_axis_name="c", subcore_axis_name="s")

    @functools.partial(
        pl.kernel,
        mesh=mesh,
        out_type=jax.ShapeDtypeStruct((C, H), jnp.float32),
        compiler_params=pltpu.CompilerParams(needs_layout_passes=False),
        scratch_types=[
            pltpu.VMEM((H,), jnp.float32),
            pltpu.VMEM((_NBUF, _CHUNK), jnp.int32),
            pltpu.VMEM((_NBUF, _CHUNK), jnp.float32),
            pltpu.SemaphoreType.DMA,
            pltpu.SemaphoreType.DMA,
            pltpu.SemaphoreType.DMA,
            pltpu.SemaphoreType.DMA,
        ],
    )
    def k(data_hbm, idx_hbm, out_hbm, accum, idxb, datab, si0, si1, sd0, sd1):
        ch = lax.axis_index("s") * _NC + lax.axis_index("c")
        sems_i = (si0, si1)
        sems_d = (sd0, sd1)

        def start(g, b):
            off = g * _CHUNK
            pltpu.async_copy(idx_hbm.at[pl.ds(off, _CHUNK)], idxb.at[b], sems_i[b])
            pltpu.async_copy(
                data_hbm.at[ch, pl.ds(off, _CHUNK)], datab.at[b], sems_d[b]
            )

        def wait(g, b):
            off = g * _CHUNK
            pltpu.make_async_copy(
                idx_hbm.at[pl.ds(off, _CHUNK)], idxb.at[b], sems_i[b]
            ).wait()
            pltpu.make_async_copy(
                data_hbm.at[ch, pl.ds(off, _CHUNK)], datab.at[b], sems_d[b]
            ).wait()

        start(0, 0)
        start(1, 1)

        zeros = jnp.zeros((_L,), jnp.float32)

        def zbody(i, carry):
            accum[pl.ds(i * _L, _L)] = zeros
            return carry

        lax.fori_loop(0, H // _L, zbody, 0)

        def outer(gg, carry):
            for b in range(_NBUF):
                g = gg * _NBUF + b
                wait(g, b)

                def ibody(i, icarry):
                    base = i * (_UNROLL * _L)
                    for u in range(_UNROLL):
                        vi = idxb[b, pl.ds(base + u * _L, _L)]
                        vd = datab[b, pl.ds(base + u * _L, _L)]
                        plsc.addupdate_scatter(accum, [vi], vd)
                    return icarry

                lax.fori_loop(0, _CHUNK // _L // _UNROLL, ibody, 0)

                @pl.when(g + _NBUF < n_chunks)
                def _():
                    start(g + _NBUF, b)

            return carry

        lax.fori_loop(0, n_chunks // _NBUF, outer, 0)
        pltpu.sync_copy(accum, out_hbm.at[ch])

    return k(data2, idx)


def kernel(data_in, octree):
    C, K, H = data_in.shape
    idx = octree.T.reshape(K * H)
    data2 = data_in.reshape(C, K * H)
    return _col2octree_sc(data2, idx, C, K, H)


# speedup vs baseline: 5.1010x; 1.0868x over previous
import functools

import jax
import jax.numpy as jnp
from jax import lax
from jax.experimental import pallas as pl
from jax.experimental.pallas import tpu as pltpu
from jax.experimental.pallas import tpu_sc as plsc

_INFO = plsc.get_sparse_core_info()
_NC, _NS, _L = _INFO.num_cores, _INFO.num_subcores, _INFO.num_lanes

_B = 512
_NBUF = 2


@functools.partial(jax.jit, static_argnums=(2, 3, 4))
def _col2octree_sc(data_in, octree_flat, C, K, H):
    n_chunks = H // _B

    mesh = plsc.VectorSubcoreMesh(core_axis_name="c", subcore_axis_name="s")

    @functools.partial(
        pl.kernel,
        mesh=mesh,
        out_type=jax.ShapeDtypeStruct((C, H), jnp.float32),
        compiler_params=pltpu.CompilerParams(needs_layout_passes=False),
        scratch_types=[
            pltpu.VMEM((H,), jnp.float32),
            pltpu.VMEM((_B * K,), jnp.int32),
            pltpu.VMEM((_B * K,), jnp.int32),
            pltpu.VMEM((K, _B), jnp.float32),
            pltpu.VMEM((K, _B), jnp.float32),
            pltpu.SemaphoreType.DMA,
            pltpu.SemaphoreType.DMA,
            pltpu.SemaphoreType.DMA,
            pltpu.SemaphoreType.DMA,
        ],
    )
    def k(data_hbm, idx_hbm, out_hbm, accum, idxb0, idxb1, datab0, datab1,
          si0, si1, sd0, sd1):
        ch = lax.axis_index("s") * _NC + lax.axis_index("c")
        idxbs = (idxb0, idxb1)
        databs = (datab0, datab1)
        sems_i = (si0, si1)
        sems_d = (sd0, sd1)

        def start(g, b):
            pltpu.async_copy(
                idx_hbm.at[pl.ds(g * (_B * K), _B * K)], idxbs[b], sems_i[b]
            )
            pltpu.async_copy(
                data_hbm.at[ch, :, pl.ds(g * _B, _B)], databs[b], sems_d[b]
            )

        def wait(g, b):
            pltpu.make_async_copy(
                idx_hbm.at[pl.ds(g * (_B * K), _B * K)], idxbs[b], sems_i[b]
            ).wait()
            pltpu.make_async_copy(
                data_hbm.at[ch, :, pl.ds(g * _B, _B)], databs[b], sems_d[b]
            ).wait()

        start(0, 0)
        start(1, 1)

        zeros = jnp.zeros((_L,), jnp.float32)

        def zbody(i, carry):
            accum[pl.ds(i * _L, _L)] = zeros
            return carry

        lax.fori_loop(0, H // _L, zbody, 0)

        viota_k = lax.iota(jnp.int32, _L) * K

        def outer(gg, carry):
            for b in range(_NBUF):
                g = gg * _NBUF + b
                wait(g, b)
                idx_flat = idxbs[b]
                datab = databs[b]

                def ibody(i, icarry):
                    base = i * (_L * K)
                    for kk in range(K):
                        vi = plsc.load_gather(idx_flat, [viota_k + (base + kk)])
                        vd = datab[kk, pl.ds(i * _L, _L)]
                        plsc.addupdate_scatter(accum, [vi], vd)
                    return icarry

                lax.fori_loop(0, _B // _L, ibody, 0)

                @pl.when(g + _NBUF < n_chunks)
                def _():
                    start(g + _NBUF, b)

            return carry

        lax.fori_loop(0, n_chunks // _NBUF, outer, 0)
        pltpu.sync_copy(accum, out_hbm.at[ch])

    return k(data_in, octree_flat)


def kernel(data_in, octree):
    C, K, H = data_in.shape
    return _col2octree_sc(data_in, octree.reshape(H * K), C, K, H)


# speedup vs baseline: 8.6963x; 1.7048x over previous
import functools

import jax
import jax.numpy as jnp
from jax import lax
from jax.experimental import pallas as pl
from jax.experimental.pallas import tpu as pltpu
from jax.experimental.pallas import tpu_sc as plsc

_INFO = plsc.get_sparse_core_info()
_NC, _NS, _L = _INFO.num_cores, _INFO.num_subcores, _INFO.num_lanes

_B = 512
_NBUF = 2


@functools.partial(jax.jit, static_argnums=(2, 3, 4))
def _col2octree_sc(data_in, octree_flat, C, K, H):
    n_chunks = H // _B

    mesh = plsc.VectorSubcoreMesh(core_axis_name="c", subcore_axis_name="s")

    @functools.partial(
        pl.kernel,
        mesh=mesh,
        out_type=jax.ShapeDtypeStruct((C, H), jnp.float32),
        compiler_params=pltpu.CompilerParams(needs_layout_passes=False),
        scratch_types=[
            pltpu.VMEM((H,), jnp.float32),
            pltpu.VMEM((_B * K,), jnp.int32),
            pltpu.VMEM((_B * K,), jnp.int32),
            pltpu.VMEM((K, _B), jnp.float32),
            pltpu.VMEM((K, _B), jnp.float32),
            pltpu.SemaphoreType.DMA,
            pltpu.SemaphoreType.DMA,
            pltpu.SemaphoreType.DMA,
            pltpu.SemaphoreType.DMA,
        ],
    )
    def k(data_hbm, idx_hbm, out_hbm, accum, idxb0, idxb1, datab0, datab1,
          si0, si1, sd0, sd1):
        ch = lax.axis_index("s") * _NC + lax.axis_index("c")
        idxbs = (idxb0, idxb1)
        databs = (datab0, datab1)
        sems_i = (si0, si1)
        sems_d = (sd0, sd1)

        def start(g, b):
            pltpu.async_copy(
                idx_hbm.at[pl.ds(g * (_B * K), _B * K)], idxbs[b], sems_i[b]
            )
            pltpu.async_copy(
                data_hbm.at[ch, :, pl.ds(g * _B, _B)], databs[b], sems_d[b]
            )

        def wait(g, b):
            pltpu.make_async_copy(
                idx_hbm.at[pl.ds(g * (_B * K), _B * K)], idxbs[b], sems_i[b]
            ).wait()
            pltpu.make_async_copy(
                data_hbm.at[ch, :, pl.ds(g * _B, _B)], databs[b], sems_d[b]
            ).wait()

        start(0, 0)
        start(1, 1)

        zeros = jnp.zeros((_L,), jnp.float32)

        def zbody(i, carry):
            accum[pl.ds(i * _L, _L)] = zeros
            return carry

        lax.fori_loop(0, H // _L, zbody, 0)

        viota_k = lax.iota(jnp.int32, _L) * K

        def outer(gg, carry):
            for b in range(_NBUF):
                g = gg * _NBUF + b
                wait(g, b)
                idx_flat = idxbs[b]
                datab = databs[b]

                def ibody(i, icarry):
                    base = i * (_L * K)
                    for kb in range(0, K, 9):
                        kks = range(kb, min(kb + 9, K))
                        vis = [
                            plsc.load_gather(idx_flat, [viota_k + (base + kk)])
                            for kk in kks
                        ]
                        vds = [datab[kk, pl.ds(i * _L, _L)] for kk in kks]
                        for vi, vd in zip(vis, vds):
                            plsc.addupdate_scatter(accum, [vi], vd)
                    return icarry

                lax.fori_loop(0, _B // _L, ibody, 0)

                @pl.when(g + _NBUF < n_chunks)
                def _():
                    start(g + _NBUF, b)

            return carry

        lax.fori_loop(0, n_chunks // _NBUF, outer, 0)
        pltpu.sync_copy(accum, out_hbm.at[ch])

    return k(data_in, octree_flat)


def kernel(data_in, octree):
    C, K, H = data_in.shape
    return _col2octree_sc(data_in, octree.reshape(H * K), C, K, H)


# speedup vs baseline: 14.0521x; 1.6159x over previous
import functools

import jax
import jax.numpy as jnp
from jax import lax
from jax.experimental import pallas as pl
from jax.experimental.pallas import tpu as pltpu
from jax.experimental.pallas import tpu_sc as plsc

_INFO = plsc.get_sparse_core_info()
_NC, _NS, _L = _INFO.num_cores, _INFO.num_subcores, _INFO.num_lanes

_B = 512
_NBUF = 2


@functools.partial(jax.jit, static_argnums=(2, 3, 4))
def _col2octree_sc(data_t, octree_t, C, K, H):
    n_chunks = H // _B

    mesh = plsc.VectorSubcoreMesh(core_axis_name="c", subcore_axis_name="s")

    @functools.partial(
        pl.kernel,
        mesh=mesh,
        out_type=jax.ShapeDtypeStruct((C, H), jnp.float32),
        compiler_params=pltpu.CompilerParams(needs_layout_passes=False),
        scratch_types=[
            pltpu.VMEM((H,), jnp.float32),
            pltpu.VMEM((K, _B), jnp.int32),
            pltpu.VMEM((K, _B), jnp.int32),
            pltpu.VMEM((K, _B), jnp.float32),
            pltpu.VMEM((K, _B), jnp.float32),
            pltpu.SemaphoreType.DMA,
            pltpu.SemaphoreType.DMA,
            pltpu.SemaphoreType.DMA,
            pltpu.SemaphoreType.DMA,
        ],
    )
    def k(data_hbm, idx_hbm, out_hbm, accum, idxb0, idxb1, datab0, datab1,
          si0, si1, sd0, sd1):
        ch = lax.axis_index("s") * _NC + lax.axis_index("c")
        idxbs = (idxb0, idxb1)
        databs = (datab0, datab1)
        sems_i = (si0, si1)
        sems_d = (sd0, sd1)

        def start(g, b):
            pltpu.async_copy(
                idx_hbm.at[:, pl.ds(g * _B, _B)], idxbs[b], sems_i[b]
            )
            pltpu.async_copy(
                data_hbm.at[:, ch, pl.ds(g * _B, _B)], databs[b], sems_d[b]
            )

        def wait(g, b):
            pltpu.make_async_copy(
                idx_hbm.at[:, pl.ds(g * _B, _B)], idxbs[b], sems_i[b]
            ).wait()
            pltpu.make_async_copy(
                data_hbm.at[:, ch, pl.ds(g * _B, _B)], databs[b], sems_d[b]
            ).wait()

        start(0, 0)
        start(1, 1)

        zeros = jnp.zeros((_L,), jnp.float32)

        def zbody(i, carry):
            accum[pl.ds(i * _L, _L)] = zeros
            return carry

        lax.fori_loop(0, H // _L, zbody, 0)

        def outer(gg, carry):
            for b in range(_NBUF):
                g = gg * _NBUF + b
                wait(g, b)
                idxb = idxbs[b]
                datab = databs[b]

                def ibody(i, icarry):
                    sl = pl.ds(i * _L, _L)
                    for kb in range(0, K, 9):
                        kks = range(kb, min(kb + 9, K))
                        vis = [idxb[kk, sl] for kk in kks]
                        vds = [datab[kk, sl] for kk in kks]
                        for vi, vd in zip(vis, vds):
                            plsc.addupdate_scatter(accum, [vi], vd)
                    return icarry

                lax.fori_loop(0, _B // _L, ibody, 0)

                @pl.when(g + _NBUF < n_chunks)
                def _():
                    start(g + _NBUF, b)

            return carry

        lax.fori_loop(0, n_chunks // _NBUF, outer, 0)
        pltpu.sync_copy(accum, out_hbm.at[ch])

    return k(data_t, octree_t)


def kernel(data_in, octree):
    C, K, H = data_in.shape
    data_t = jnp.transpose(data_in, (1, 0, 2))
    octree_t = octree.T
    return _col2octree_sc(data_t, octree_t, C, K, H)
